# initial kernel scaffold (unmeasured)
import jax
import jax.numpy as jnp
from jax import lax
from jax.experimental import pallas as pl
from jax.experimental.pallas import tpu as pltpu

N_DEV = 4


def kernel(partial, resid, gamma):
    m, d = resid.shape
    ch = m // N_DEV
    x = partial.reshape(m, d)
    gamma2 = gamma.reshape(1, d)

    def body(
        x_ref,
        resid_ref,
        gamma_ref,
        out_ref,
        acc_ref,
        rs_recv_ref,
        rs_send_sems,
        rs_recv_sems,
        ag_send_sems,
        ag_recv_sems,
    ):
        my = lax.axis_index("i")
        right = lax.rem(my + 1, N_DEV)
        left = lax.rem(my + 3, N_DEV)

        barrier_sem = pltpu.get_barrier_semaphore()
        for nbr in (left, right):
            pl.semaphore_signal(
                barrier_sem,
                inc=1,
                device_id=(nbr,),
                device_id_type=pl.DeviceIdType.MESH,
            )
        pl.semaphore_wait(barrier_sem, 2)

        for s in range(N_DEV - 1):
            c_send = lax.rem(my + (N_DEV - 1 - s), N_DEV)
            c_recv = lax.rem(my + (N_DEV - 2 - s), N_DEV)
            if s == 0:
                src = x_ref.at[pl.ds(c_send * ch, ch), :]
            else:
                src = acc_ref.at[(s - 1) % 2]
            rdma = pltpu.make_async_remote_copy(
                src_ref=src,
                dst_ref=rs_recv_ref.at[s],
                send_sem=rs_send_sems.at[s],
                recv_sem=rs_recv_sems.at[s],
                device_id=(right,),
                device_id_type=pl.DeviceIdType.MESH,
            )
            rdma.start()
            rdma.wait()
            summed = rs_recv_ref[s] + x_ref[pl.ds(c_recv * ch, ch), :]
            if s < N_DEV - 2:
                acc_ref[s % 2] = summed
            else:
                y = summed + resid_ref[pl.ds(my * ch, ch), :]
                rms = jnp.sqrt(jnp.mean(y * y, axis=-1, keepdims=True) + 1e-6)
                out_ref[pl.ds(my * ch, ch), :] = (y / rms) * gamma_ref[:, :]

        for h in range(N_DEV - 1):
            c = lax.rem(my + (N_DEV - h), N_DEV)
            rdma = pltpu.make_async_remote_copy(
                src_ref=out_ref.at[pl.ds(c * ch, ch), :],
                dst_ref=out_ref.at[pl.ds(c * ch, ch), :],
                send_sem=ag_send_sems.at[h],
                recv_sem=ag_recv_sems.at[h],
                device_id=(right,),
                device_id_type=pl.DeviceIdType.MESH,
            )
            rdma.start()
            rdma.wait()

    return pl.pallas_call(
        body,
        out_shape=jax.ShapeDtypeStruct((m, d), jnp.float32),
        in_specs=[
            pl.BlockSpec(memory_space=pltpu.VMEM),
            pl.BlockSpec(memory_space=pltpu.VMEM),
            pl.BlockSpec(memory_space=pltpu.VMEM),
        ],
        out_specs=pl.BlockSpec(memory_space=pltpu.VMEM),
        scratch_shapes=[
            pltpu.VMEM((2, ch, d), jnp.float32),
            pltpu.VMEM((N_DEV - 1, ch, d), jnp.float32),
            pltpu.SemaphoreType.DMA((N_DEV - 1,)),
            pltpu.SemaphoreType.DMA((N_DEV - 1,)),
            pltpu.SemaphoreType.DMA((N_DEV - 1,)),
            pltpu.SemaphoreType.DMA((N_DEV - 1,)),
        ],
        compiler_params=pltpu.CompilerParams(
            collective_id=0,
            vmem_limit_bytes=128 * 1024 * 1024,
        ),
    )(x, resid, gamma2)


# baseline (device time: 308771 ns/iter reference)
import jax
import jax.numpy as jnp
from jax import lax
from jax.experimental import pallas as pl
from jax.experimental.pallas import tpu as pltpu

N_DEV = 4


def kernel(partial, resid, gamma):
    m, d = resid.shape
    ch = m // N_DEV
    x = partial.reshape(m, d)
    gamma2 = gamma.reshape(1, d)

    def body(
        x_ref,
        resid_ref,
        gamma_ref,
        out_ref,
        acc_ref,
        rs_recv_ref,
        resid_vmem,
        resid_sem,
        rs_send_sems,
        rs_recv_sems,
        ag_send_sems,
        ag_recv_sems,
    ):
        my = lax.axis_index("i")
        right = lax.rem(my + 1, N_DEV)
        left = lax.rem(my + 3, N_DEV)

        resid_copy = pltpu.make_async_copy(
            resid_ref.at[pl.ds(my * ch, ch), :], resid_vmem, resid_sem
        )
        resid_copy.start()

        barrier_sem = pltpu.get_barrier_semaphore()
        for nbr in (left, right):
            pl.semaphore_signal(
                barrier_sem,
                inc=1,
                device_id=(nbr,),
                device_id_type=pl.DeviceIdType.MESH,
            )
        pl.semaphore_wait(barrier_sem, 2)

        for s in range(N_DEV - 1):
            c_send = lax.rem(my + (N_DEV - 1 - s), N_DEV)
            c_recv = lax.rem(my + (N_DEV - 2 - s), N_DEV)
            if s == 0:
                src = x_ref.at[pl.ds(c_send * ch, ch), :]
            else:
                src = acc_ref.at[(s - 1) % 2]
            rdma = pltpu.make_async_remote_copy(
                src_ref=src,
                dst_ref=rs_recv_ref.at[s],
                send_sem=rs_send_sems.at[s],
                recv_sem=rs_recv_sems.at[s],
                device_id=(right,),
                device_id_type=pl.DeviceIdType.MESH,
            )
            rdma.start()
            rdma.wait()
            summed = rs_recv_ref[s] + x_ref[pl.ds(c_recv * ch, ch), :]
            if s < N_DEV - 2:
                acc_ref[s % 2] = summed
            else:
                resid_copy.wait()
                y = summed + resid_vmem[:, :]
                rms = jnp.sqrt(jnp.mean(y * y, axis=-1, keepdims=True) + 1e-6)
                out_ref[pl.ds(my * ch, ch), :] = (y / rms) * gamma_ref[:, :]

        for h in range(N_DEV - 1):
            c = lax.rem(my + (N_DEV - h), N_DEV)
            rdma = pltpu.make_async_remote_copy(
                src_ref=out_ref.at[pl.ds(c * ch, ch), :],
                dst_ref=out_ref.at[pl.ds(c * ch, ch), :],
                send_sem=ag_send_sems.at[h],
                recv_sem=ag_recv_sems.at[h],
                device_id=(right,),
                device_id_type=pl.DeviceIdType.MESH,
            )
            rdma.start()
            rdma.wait()

    return pl.pallas_call(
        body,
        out_shape=jax.ShapeDtypeStruct((m, d), jnp.float32),
        in_specs=[
            pl.BlockSpec(memory_space=pltpu.VMEM),
            pl.BlockSpec(memory_space=pl.ANY),
            pl.BlockSpec(memory_space=pltpu.VMEM),
        ],
        out_specs=pl.BlockSpec(memory_space=pltpu.VMEM),
        scratch_shapes=[
            pltpu.VMEM((2, ch, d), jnp.float32),
            pltpu.VMEM((N_DEV - 1, ch, d), jnp.float32),
            pltpu.VMEM((ch, d), jnp.float32),
            pltpu.SemaphoreType.DMA,
            pltpu.SemaphoreType.DMA((N_DEV - 1,)),
            pltpu.SemaphoreType.DMA((N_DEV - 1,)),
            pltpu.SemaphoreType.DMA((N_DEV - 1,)),
            pltpu.SemaphoreType.DMA((N_DEV - 1,)),
        ],
        compiler_params=pltpu.CompilerParams(
            collective_id=0,
            vmem_limit_bytes=128 * 1024 * 1024,
        ),
    )(x, resid, gamma2)


# device time: 174102 ns/iter; 1.7735x vs baseline; 1.7735x over previous
import jax
import jax.numpy as jnp
from jax import lax
from jax.experimental import pallas as pl
from jax.experimental.pallas import tpu as pltpu

N_DEV = 4


def kernel(partial, resid, gamma):
    m, d = resid.shape
    ch = m // N_DEV
    hw = d // 2
    x = partial.reshape(m, d)
    gamma2 = gamma.reshape(1, d)

    def body(
        x_ref,
        resid_ref,
        gamma_ref,
        out_ref,
        accR_ref,
        accL_ref,
        rsR_ref,
        rsL_ref,
        resid_vmem,
        resid_sem,
        rsR_send_sems,
        rsR_recv_sems,
        rsL_send_sems,
        rsL_recv_sems,
        agR_send_sems,
        agR_recv_sems,
        agL_send_sems,
        agL_recv_sems,
    ):
        my = lax.axis_index("i")
        right = lax.rem(my + 1, N_DEV)
        left = lax.rem(my + 3, N_DEV)

        resid_copy = pltpu.make_async_copy(
            resid_ref.at[pl.ds(my * ch, ch), :], resid_vmem, resid_sem
        )
        resid_copy.start()

        barrier_sem = pltpu.get_barrier_semaphore()
        for nbr in (left, right):
            pl.semaphore_signal(
                barrier_sem,
                inc=1,
                device_id=(nbr,),
                device_id_type=pl.DeviceIdType.MESH,
            )
        pl.semaphore_wait(barrier_sem, 2)

        for s in range(N_DEV - 1):
            cR_send = lax.rem(my + (N_DEV - 1 - s), N_DEV)
            cR_recv = lax.rem(my + (N_DEV - 2 - s), N_DEV)
            cL_send = lax.rem(my + 1 + s, N_DEV)
            cL_recv = lax.rem(my + 2 + s, N_DEV)
            if s == 0:
                srcR = x_ref.at[pl.ds(cR_send * ch, ch), pl.ds(0, hw)]
                srcL = x_ref.at[pl.ds(cL_send * ch, ch), pl.ds(hw, hw)]
            else:
                srcR = accR_ref.at[(s - 1) % 2]
                srcL = accL_ref.at[(s - 1) % 2]
            rdmaR = pltpu.make_async_remote_copy(
                src_ref=srcR,
                dst_ref=rsR_ref.at[s],
                send_sem=rsR_send_sems.at[s],
                recv_sem=rsR_recv_sems.at[s],
                device_id=(right,),
                device_id_type=pl.DeviceIdType.MESH,
            )
            rdmaL = pltpu.make_async_remote_copy(
                src_ref=srcL,
                dst_ref=rsL_ref.at[s],
                send_sem=rsL_send_sems.at[s],
                recv_sem=rsL_recv_sems.at[s],
                device_id=(left,),
                device_id_type=pl.DeviceIdType.MESH,
            )
            rdmaR.start()
            rdmaL.start()
            rdmaR.wait()
            rdmaL.wait()
            summedR = rsR_ref[s] + x_ref[pl.ds(cR_recv * ch, ch), pl.ds(0, hw)]
            summedL = rsL_ref[s] + x_ref[pl.ds(cL_recv * ch, ch), pl.ds(hw, hw)]
            if s < N_DEV - 2:
                accR_ref[s % 2] = summedR
                accL_ref[s % 2] = summedL
            else:
                resid_copy.wait()
                yR = summedR + resid_vmem[:, 0:hw]
                yL = summedL + resid_vmem[:, hw : 2 * hw]
                ssq = jnp.sum(yR * yR, axis=-1, keepdims=True) + jnp.sum(
                    yL * yL, axis=-1, keepdims=True
                )
                inv = lax.rsqrt(ssq / d + 1e-6)
                out_ref[pl.ds(my * ch, ch), pl.ds(0, hw)] = (
                    yR * inv * gamma_ref[:, 0:hw]
                )
                out_ref[pl.ds(my * ch, ch), pl.ds(hw, hw)] = (
                    yL * inv * gamma_ref[:, hw : 2 * hw]
                )

        for h in range(N_DEV - 1):
            cR = lax.rem(my + (N_DEV - h), N_DEV)
            cL = lax.rem(my + h, N_DEV)
            rdmaR = pltpu.make_async_remote_copy(
                src_ref=out_ref.at[pl.ds(cR * ch, ch), pl.ds(0, hw)],
                dst_ref=out_ref.at[pl.ds(cR * ch, ch), pl.ds(0, hw)],
                send_sem=agR_send_sems.at[h],
                recv_sem=agR_recv_sems.at[h],
                device_id=(right,),
                device_id_type=pl.DeviceIdType.MESH,
            )
            rdmaL = pltpu.make_async_remote_copy(
                src_ref=out_ref.at[pl.ds(cL * ch, ch), pl.ds(hw, hw)],
                dst_ref=out_ref.at[pl.ds(cL * ch, ch), pl.ds(hw, hw)],
                send_sem=agL_send_sems.at[h],
                recv_sem=agL_recv_sems.at[h],
                device_id=(left,),
                device_id_type=pl.DeviceIdType.MESH,
            )
            rdmaR.start()
            rdmaL.start()
            rdmaR.wait()
            rdmaL.wait()

    return pl.pallas_call(
        body,
        out_shape=jax.ShapeDtypeStruct((m, d), jnp.float32),
        in_specs=[
            pl.BlockSpec(memory_space=pltpu.VMEM),
            pl.BlockSpec(memory_space=pl.ANY),
            pl.BlockSpec(memory_space=pltpu.VMEM),
        ],
        out_specs=pl.BlockSpec(memory_space=pltpu.VMEM),
        scratch_shapes=[
            pltpu.VMEM((2, ch, hw), jnp.float32),
            pltpu.VMEM((2, ch, hw), jnp.float32),
            pltpu.VMEM((N_DEV - 1, ch, hw), jnp.float32),
            pltpu.VMEM((N_DEV - 1, ch, hw), jnp.float32),
            pltpu.VMEM((ch, d), jnp.float32),
            pltpu.SemaphoreType.DMA,
            pltpu.SemaphoreType.DMA((N_DEV - 1,)),
            pltpu.SemaphoreType.DMA((N_DEV - 1,)),
            pltpu.SemaphoreType.DMA((N_DEV - 1,)),
            pltpu.SemaphoreType.DMA((N_DEV - 1,)),
            pltpu.SemaphoreType.DMA((N_DEV - 1,)),
            pltpu.SemaphoreType.DMA((N_DEV - 1,)),
            pltpu.SemaphoreType.DMA((N_DEV - 1,)),
            pltpu.SemaphoreType.DMA((N_DEV - 1,)),
        ],
        compiler_params=pltpu.CompilerParams(
            collective_id=0,
            vmem_limit_bytes=128 * 1024 * 1024,
        ),
    )(x, resid, gamma2)


# device time: 160163 ns/iter; 1.9279x vs baseline; 1.0870x over previous
import jax
import jax.numpy as jnp
from jax import lax
from jax.experimental import pallas as pl
from jax.experimental.pallas import tpu as pltpu

N_DEV = 4
K = 2


def kernel(partial, resid, gamma):
    m, d = resid.shape
    ch = m // N_DEV
    hw = d // 2
    seg = ch // K
    x = partial.reshape(m, d)
    gamma2 = gamma.reshape(1, d)

    def body(
        x_ref,
        resid_ref,
        gamma_ref,
        out_ref,
        accR_ref,
        accL_ref,
        rsR_ref,
        rsL_ref,
        stage_a,
        stage_b,
        stage_c,
        resid_vmem,
        stage_sems,
        rsR_send_sems,
        rsR_recv_sems,
        rsL_send_sems,
        rsL_recv_sems,
        agR_send_sems,
        agR_recv_sems,
        agL_send_sems,
        agL_recv_sems,
    ):
        my = lax.axis_index("i")
        right = lax.rem(my + 1, N_DEV)
        left = lax.rem(my + 3, N_DEV)
        lo = pl.ds(0, hw)
        hi = pl.ds(hw, hw)

        def rows(c, k):
            return pl.ds(c * ch + k * seg, seg)

        c_a = lax.rem(my + 2, N_DEV)
        c_b_lo = lax.rem(my + 1, N_DEV)
        c_b_hi = lax.rem(my + 3, N_DEV)
        stages = [
            pltpu.make_async_copy(
                x_ref.at[pl.ds(c_a * ch, ch), :], stage_a, stage_sems.at[0]
            ),
            pltpu.make_async_copy(
                x_ref.at[pl.ds(c_b_lo * ch, ch), lo],
                stage_b.at[:, lo],
                stage_sems.at[1],
            ),
            pltpu.make_async_copy(
                x_ref.at[pl.ds(c_b_hi * ch, ch), hi],
                stage_b.at[:, hi],
                stage_sems.at[2],
            ),
            pltpu.make_async_copy(
                x_ref.at[pl.ds(my * ch, ch), :], stage_c, stage_sems.at[3]
            ),
            pltpu.make_async_copy(
                resid_ref.at[pl.ds(my * ch, ch), :],
                resid_vmem,
                stage_sems.at[4],
            ),
        ]
        for c in stages:
            c.start()

        barrier_sem = pltpu.get_barrier_semaphore()
        for nbr in (left, right):
            pl.semaphore_signal(
                barrier_sem,
                inc=1,
                device_id=(nbr,),
                device_id_type=pl.DeviceIdType.MESH,
            )
        pl.semaphore_wait(barrier_sem, 2)

        started = []

        def rs_rdma(s, k, dirn):
            if dirn == "R":
                c_send = lax.rem(my + (N_DEV - 1 - s), N_DEV)
                src = (
                    x_ref.at[rows(c_send, k), lo]
                    if s == 0
                    else accR_ref.at[s - 1, pl.ds(k * seg, seg), :]
                )
                return pltpu.make_async_remote_copy(
                    src_ref=src,
                    dst_ref=rsR_ref.at[s, pl.ds(k * seg, seg), :],
                    send_sem=rsR_send_sems.at[s, k],
                    recv_sem=rsR_recv_sems.at[s, k],
                    device_id=(right,),
                    device_id_type=pl.DeviceIdType.MESH,
                )
            c_send = lax.rem(my + 1 + s, N_DEV)
            src = (
                x_ref.at[rows(c_send, k), hi]
                if s == 0
                else accL_ref.at[s - 1, pl.ds(k * seg, seg), :]
            )
            return pltpu.make_async_remote_copy(
                src_ref=src,
                dst_ref=rsL_ref.at[s, pl.ds(k * seg, seg), :],
                send_sem=rsL_send_sems.at[s, k],
                recv_sem=rsL_recv_sems.at[s, k],
                device_id=(left,),
                device_id_type=pl.DeviceIdType.MESH,
            )

        def ag_rdma(h, k, dirn):
            if dirn == "R":
                c = lax.rem(my + (N_DEV - h), N_DEV)
                ref = out_ref.at[rows(c, k), lo]
                return pltpu.make_async_remote_copy(
                    src_ref=ref,
                    dst_ref=ref,
                    send_sem=agR_send_sems.at[h, k],
                    recv_sem=agR_recv_sems.at[h, k],
                    device_id=(right,),
                    device_id_type=pl.DeviceIdType.MESH,
                )
            c = lax.rem(my + h, N_DEV)
            ref = out_ref.at[rows(c, k), hi]
            return pltpu.make_async_remote_copy(
                src_ref=ref,
                dst_ref=ref,
                send_sem=agL_send_sems.at[h, k],
                recv_sem=agL_recv_sems.at[h, k],
                device_id=(left,),
                device_id_type=pl.DeviceIdType.MESH,
            )

        def start(rdma):
            rdma.start()
            started.append(rdma)
            return rdma

        rs_in_flight = {}
        for k in range(K):
            for dirn in ("R", "L"):
                rs_in_flight[(0, k, dirn)] = start(rs_rdma(0, k, dirn))

        x_stage = {0: stage_a, 1: stage_b, 2: stage_c}
        ag_in_flight = {}
        for s in range(N_DEV - 1):
            for k in range(K):
                if k == 0:
                    for idx in {0: [0], 1: [1, 2], 2: [3, 4]}[s]:
                        stages[idx].wait()
                for dirn in ("R", "L"):
                    rs_in_flight[(s, k, dirn)].wait_recv()
                summedR = (
                    rsR_ref[s, pl.ds(k * seg, seg), :]
                    + x_stage[s][pl.ds(k * seg, seg), lo]
                )
                summedL = (
                    rsL_ref[s, pl.ds(k * seg, seg), :]
                    + x_stage[s][pl.ds(k * seg, seg), hi]
                )
                if s < N_DEV - 2:
                    accR_ref[s, pl.ds(k * seg, seg), :] = summedR
                    accL_ref[s, pl.ds(k * seg, seg), :] = summedL
                    for dirn in ("R", "L"):
                        rs_in_flight[(s + 1, k, dirn)] = start(
                            rs_rdma(s + 1, k, dirn)
                        )
                else:
                    y_lo = summedR + resid_vmem[pl.ds(k * seg, seg), lo]
                    y_hi = summedL + resid_vmem[pl.ds(k * seg, seg), hi]
                    ssq = jnp.sum(y_lo * y_lo, axis=-1, keepdims=True) + jnp.sum(
                        y_hi * y_hi, axis=-1, keepdims=True
                    )
                    inv = lax.rsqrt(ssq / d + 1e-6)
                    out_ref[rows(my, k), lo] = y_lo * inv * gamma_ref[:, lo]
                    out_ref[rows(my, k), hi] = y_hi * inv * gamma_ref[:, hi]
                    for dirn in ("R", "L"):
                        ag_in_flight[(0, k, dirn)] = start(ag_rdma(0, k, dirn))

        for h in range(N_DEV - 1):
            for k in range(K):
                for dirn in ("R", "L"):
                    ag_in_flight[(h, k, dirn)].wait_recv()
                    if h < N_DEV - 2:
                        ag_in_flight[(h + 1, k, dirn)] = start(
                            ag_rdma(h + 1, k, dirn)
                        )

        for rdma in started:
            rdma.wait_send()

    return pl.pallas_call(
        body,
        out_shape=jax.ShapeDtypeStruct((m, d), jnp.float32),
        in_specs=[
            pl.BlockSpec(memory_space=pl.ANY),
            pl.BlockSpec(memory_space=pl.ANY),
            pl.BlockSpec(memory_space=pltpu.VMEM),
        ],
        out_specs=pl.BlockSpec(memory_space=pltpu.VMEM),
        scratch_shapes=[
            pltpu.VMEM((2, ch, hw), jnp.float32),
            pltpu.VMEM((2, ch, hw), jnp.float32),
            pltpu.VMEM((N_DEV - 1, ch, hw), jnp.float32),
            pltpu.VMEM((N_DEV - 1, ch, hw), jnp.float32),
            pltpu.VMEM((ch, d), jnp.float32),
            pltpu.VMEM((ch, d), jnp.float32),
            pltpu.VMEM((ch, d), jnp.float32),
            pltpu.VMEM((ch, d), jnp.float32),
            pltpu.SemaphoreType.DMA((5,)),
            pltpu.SemaphoreType.DMA((N_DEV - 1, K)),
            pltpu.SemaphoreType.DMA((N_DEV - 1, K)),
            pltpu.SemaphoreType.DMA((N_DEV - 1, K)),
            pltpu.SemaphoreType.DMA((N_DEV - 1, K)),
            pltpu.SemaphoreType.DMA((N_DEV - 1, K)),
            pltpu.SemaphoreType.DMA((N_DEV - 1, K)),
            pltpu.SemaphoreType.DMA((N_DEV - 1, K)),
            pltpu.SemaphoreType.DMA((N_DEV - 1, K)),
        ],
        compiler_params=pltpu.CompilerParams(
            collective_id=0,
            vmem_limit_bytes=128 * 1024 * 1024,
        ),
    )(x, resid, gamma2)


# device time: 155674 ns/iter; 1.9834x vs baseline; 1.0288x over previous
import jax
import jax.numpy as jnp
from jax import lax
from jax.experimental import pallas as pl
from jax.experimental.pallas import tpu as pltpu

N_DEV = 4
K = 2


def kernel(partial, resid, gamma):
    m, d = resid.shape
    ch = m // N_DEV
    hw = d // 2
    seg = ch // K
    x = partial.reshape(m, d)
    gamma2 = gamma.reshape(1, d)

    def body(
        x_ref,
        resid_ref,
        gamma_ref,
        out_ref,
        accR_ref,
        accL_ref,
        rsR_ref,
        rsL_ref,
        agR_ref,
        agL_ref,
        normR_ref,
        normL_ref,
        stage_a,
        stage_b,
        stage_c,
        resid_vmem,
        stage_sems,
        out_sems,
        rsR_send_sems,
        rsR_recv_sems,
        rsL_send_sems,
        rsL_recv_sems,
        agR_send_sems,
        agR_recv_sems,
        agL_send_sems,
        agL_recv_sems,
    ):
        my = lax.axis_index("i")
        right = lax.rem(my + 1, N_DEV)
        left = lax.rem(my + 3, N_DEV)
        lo = pl.ds(0, hw)
        hi = pl.ds(hw, hw)

        def rows(c, k):
            return pl.ds(c * ch + k * seg, seg)

        def segr(k):
            return pl.ds(k * seg, seg)

        c_a = lax.rem(my + 2, N_DEV)
        c_b_lo = lax.rem(my + 1, N_DEV)
        c_b_hi = lax.rem(my + 3, N_DEV)
        stages = [
            pltpu.make_async_copy(
                x_ref.at[pl.ds(c_a * ch, ch), :], stage_a, stage_sems.at[0]
            ),
            pltpu.make_async_copy(
                x_ref.at[pl.ds(c_b_lo * ch, ch), lo],
                stage_b.at[:, lo],
                stage_sems.at[1],
            ),
            pltpu.make_async_copy(
                x_ref.at[pl.ds(c_b_hi * ch, ch), hi],
                stage_b.at[:, hi],
                stage_sems.at[2],
            ),
            pltpu.make_async_copy(
                x_ref.at[pl.ds(my * ch, ch), :], stage_c, stage_sems.at[3]
            ),
            pltpu.make_async_copy(
                resid_ref.at[pl.ds(my * ch, ch), :],
                resid_vmem,
                stage_sems.at[4],
            ),
        ]
        for c in stages:
            c.start()

        barrier_sem = pltpu.get_barrier_semaphore()
        for nbr in (left, right):
            pl.semaphore_signal(
                barrier_sem,
                inc=1,
                device_id=(nbr,),
                device_id_type=pl.DeviceIdType.MESH,
            )
        pl.semaphore_wait(barrier_sem, 2)

        started = []
        out_copies = []

        def rs_rdma(s, k, dirn):
            if dirn == "R":
                c_send = lax.rem(my + (N_DEV - 1 - s), N_DEV)
                src = (
                    x_ref.at[rows(c_send, k), lo]
                    if s == 0
                    else accR_ref.at[s - 1, segr(k), :]
                )
                return pltpu.make_async_remote_copy(
                    src_ref=src,
                    dst_ref=rsR_ref.at[s, segr(k), :],
                    send_sem=rsR_send_sems.at[s, k],
                    recv_sem=rsR_recv_sems.at[s, k],
                    device_id=(right,),
                    device_id_type=pl.DeviceIdType.MESH,
                )
            c_send = lax.rem(my + 1 + s, N_DEV)
            src = (
                x_ref.at[rows(c_send, k), hi]
                if s == 0
                else accL_ref.at[s - 1, segr(k), :]
            )
            return pltpu.make_async_remote_copy(
                src_ref=src,
                dst_ref=rsL_ref.at[s, segr(k), :],
                send_sem=rsL_send_sems.at[s, k],
                recv_sem=rsL_recv_sems.at[s, k],
                device_id=(left,),
                device_id_type=pl.DeviceIdType.MESH,
            )

        def ag_rdma(h, k, dirn):
            if dirn == "R":
                src = (
                    normR_ref.at[segr(k), :]
                    if h == 0
                    else agR_ref.at[h - 1, segr(k), :]
                )
                return pltpu.make_async_remote_copy(
                    src_ref=src,
                    dst_ref=agR_ref.at[h, segr(k), :],
                    send_sem=agR_send_sems.at[h, k],
                    recv_sem=agR_recv_sems.at[h, k],
                    device_id=(right,),
                    device_id_type=pl.DeviceIdType.MESH,
                )
            src = (
                normL_ref.at[segr(k), :]
                if h == 0
                else agL_ref.at[h - 1, segr(k), :]
            )
            return pltpu.make_async_remote_copy(
                src_ref=src,
                dst_ref=agL_ref.at[h, segr(k), :],
                send_sem=agL_send_sems.at[h, k],
                recv_sem=agL_recv_sems.at[h, k],
                device_id=(left,),
                device_id_type=pl.DeviceIdType.MESH,
            )

        def start(rdma):
            rdma.start()
            started.append(rdma)
            return rdma

        def out_copy(src_ref, c, k, half, sem):
            cp = pltpu.make_async_copy(src_ref, out_ref.at[rows(c, k), half], sem)
            cp.start()
            out_copies.append(cp)

        rs_in_flight = {}
        for k in range(K):
            for dirn in ("R", "L"):
                rs_in_flight[(0, k, dirn)] = start(rs_rdma(0, k, dirn))

        x_stage = {0: stage_a, 1: stage_b, 2: stage_c}
        ag_in_flight = {}
        for s in range(N_DEV - 1):
            for k in range(K):
                if k == 0:
                    for idx in {0: [0], 1: [1, 2], 2: [3, 4]}[s]:
                        stages[idx].wait()
                for dirn in ("R", "L"):
                    rs_in_flight[(s, k, dirn)].wait_recv()
                summedR = rsR_ref[s, segr(k), :] + x_stage[s][segr(k), lo]
                summedL = rsL_ref[s, segr(k), :] + x_stage[s][segr(k), hi]
                if s < N_DEV - 2:
                    accR_ref[s, segr(k), :] = summedR
                    accL_ref[s, segr(k), :] = summedL
                    for dirn in ("R", "L"):
                        rs_in_flight[(s + 1, k, dirn)] = start(
                            rs_rdma(s + 1, k, dirn)
                        )
                else:
                    y_lo = summedR + resid_vmem[segr(k), lo]
                    y_hi = summedL + resid_vmem[segr(k), hi]
                    ssq = jnp.sum(y_lo * y_lo, axis=-1, keepdims=True) + jnp.sum(
                        y_hi * y_hi, axis=-1, keepdims=True
                    )
                    inv = lax.rsqrt(ssq / d + 1e-6)
                    normR_ref[segr(k), :] = y_lo * inv * gamma_ref[:, lo]
                    normL_ref[segr(k), :] = y_hi * inv * gamma_ref[:, hi]
                    for dirn in ("R", "L"):
                        ag_in_flight[(0, k, dirn)] = start(ag_rdma(0, k, dirn))
                    out_copy(normR_ref.at[segr(k), :], my, k, lo, out_sems.at[3, k, 0])
                    out_copy(normL_ref.at[segr(k), :], my, k, hi, out_sems.at[3, k, 1])

        for h in range(N_DEV - 1):
            for k in range(K):
                for dirn in ("R", "L"):
                    ag_in_flight[(h, k, dirn)].wait_recv()
                    if h < N_DEV - 2:
                        ag_in_flight[(h + 1, k, dirn)] = start(
                            ag_rdma(h + 1, k, dirn)
                        )
                    if dirn == "R":
                        c_in = lax.rem(my + (N_DEV - 1 - h), N_DEV)
                        out_copy(
                            agR_ref.at[h, segr(k), :],
                            c_in,
                            k,
                            lo,
                            out_sems.at[h, k, 0],
                        )
                    else:
                        c_in = lax.rem(my + 1 + h, N_DEV)
                        out_copy(
                            agL_ref.at[h, segr(k), :],
                            c_in,
                            k,
                            hi,
                            out_sems.at[h, k, 1],
                        )

        for rdma in started:
            rdma.wait_send()
        for cp in out_copies:
            cp.wait()

    return pl.pallas_call(
        body,
        out_shape=jax.ShapeDtypeStruct((m, d), jnp.float32),
        in_specs=[
            pl.BlockSpec(memory_space=pl.ANY),
            pl.BlockSpec(memory_space=pl.ANY),
            pl.BlockSpec(memory_space=pltpu.VMEM),
        ],
        out_specs=pl.BlockSpec(memory_space=pl.ANY),
        scratch_shapes=[
            pltpu.VMEM((2, ch, hw), jnp.float32),
            pltpu.VMEM((2, ch, hw), jnp.float32),
            pltpu.VMEM((N_DEV - 1, ch, hw), jnp.float32),
            pltpu.VMEM((N_DEV - 1, ch, hw), jnp.float32),
            pltpu.VMEM((N_DEV - 1, ch, hw), jnp.float32),
            pltpu.VMEM((N_DEV - 1, ch, hw), jnp.float32),
            pltpu.VMEM((ch, hw), jnp.float32),
            pltpu.VMEM((ch, hw), jnp.float32),
            pltpu.VMEM((ch, d), jnp.float32),
            pltpu.VMEM((ch, d), jnp.float32),
            pltpu.VMEM((ch, d), jnp.float32),
            pltpu.VMEM((ch, d), jnp.float32),
            pltpu.SemaphoreType.DMA((5,)),
            pltpu.SemaphoreType.DMA((N_DEV, K, 2)),
            pltpu.SemaphoreType.DMA((N_DEV - 1, K)),
            pltpu.SemaphoreType.DMA((N_DEV - 1, K)),
            pltpu.SemaphoreType.DMA((N_DEV - 1, K)),
            pltpu.SemaphoreType.DMA((N_DEV - 1, K)),
            pltpu.SemaphoreType.DMA((N_DEV - 1, K)),
            pltpu.SemaphoreType.DMA((N_DEV - 1, K)),
            pltpu.SemaphoreType.DMA((N_DEV - 1, K)),
            pltpu.SemaphoreType.DMA((N_DEV - 1, K)),
        ],
        compiler_params=pltpu.CompilerParams(
            collective_id=0,
            vmem_limit_bytes=128 * 1024 * 1024,
        ),
    )(x, resid, gamma2)
